# baseline (speedup 1.0000x reference)
import functools

import jax
import jax.numpy as jnp
from jax import lax
from jax.experimental import pallas as pl
from jax.experimental.pallas import tpu as pltpu
from jax.experimental.pallas import tpu_sc as plsc

N_NODES = 10000
N_EDGES = 320000
D = 128

CHUNK = 128
NCORES = 2
NSUB = 16
NWORKERS = NCORES * NSUB
WCHUNKS = 80
PCHUNKS = NWORKERS * WCHUNKS
PAD_EDGES = PCHUNKS * CHUNK
SCHUNKS = 16
NSTAGES = WCHUNKS // SCHUNKS
RCHUNK = 80
NRCHUNKS = N_NODES // RCHUNK
RITERS = -(-NRCHUNKS // NSUB)


def _mm_body(x_ref, w_ref, o_ref):
    o_ref[...] = jnp.dot(x_ref[...], w_ref[...],
                         preferred_element_type=jnp.float32)


def _matmul(x, w):
    return pl.pallas_call(
        _mm_body,
        grid=(10,),
        in_specs=[
            pl.BlockSpec((N_NODES // 10, D), lambda i: (i, 0)),
            pl.BlockSpec((D, D), lambda i: (0, 0)),
        ],
        out_specs=pl.BlockSpec((N_NODES // 10, D), lambda i: (i, 0)),
        out_shape=jax.ShapeDtypeStruct((N_NODES, D), jnp.float32),
    )(x, w)


def _comb_body(p_ref, b_ref, o_ref):
    o_ref[...] = p_ref[0] + p_ref[1] + b_ref[...]


def _combine(partials, b):
    return pl.pallas_call(
        _comb_body,
        grid=(10,),
        in_specs=[
            pl.BlockSpec((2, N_NODES // 10, D), lambda i: (0, i, 0)),
            pl.BlockSpec((1, D), lambda i: (0, 0)),
        ],
        out_specs=pl.BlockSpec((N_NODES // 10, D), lambda i: (i, 0)),
        out_shape=jax.ShapeDtypeStruct((N_NODES, D), jnp.float32),
    )(partials, b)


def _scale_rows(rows_ref, vals_blk, c):
    @pl.loop(0, CHUNK // 16)
    def _(eb):
        vals16 = vals_blk[pl.ds(c, 1), pl.ds(eb * 16, 16)]
        for j in range(16):
            v = vals16[0, j]
            for g in range(D // 16):
                sl = (pl.ds(eb * 16 + j, 1), pl.ds(g * 16, 16))
                rows_ref[sl] = rows_ref[sl] * v


def _spmm(hidden, src, dst, vals):
    mesh = plsc.VectorSubcoreMesh(core_axis_name="core",
                                  subcore_axis_name="subcore")

    @functools.partial(
        pl.kernel,
        out_type=jax.ShapeDtypeStruct((NCORES, N_NODES, D), jnp.float32),
        mesh=mesh,
        scratch_types=[
            pltpu.VMEM((SCHUNKS, CHUNK), jnp.int32),
            pltpu.VMEM((SCHUNKS, CHUNK), jnp.int32),
            pltpu.VMEM((SCHUNKS, CHUNK), jnp.float32),
            pltpu.VMEM((CHUNK, D), jnp.float32),
            pltpu.VMEM((CHUNK, D), jnp.float32),
            pltpu.VMEM_SHARED((N_NODES, D), jnp.float32),
            pltpu.SemaphoreType.DMA,
            pltpu.SemaphoreType.DMA,
        ],
    )
    def spmm_kernel(hid_hbm, src_hbm, dst_hbm, val_hbm, part_hbm,
                    sidx_v, didx_v, val_v, rows0, rows1, acc,
                    sem0, sem1):
        cid = lax.axis_index("core")
        tid = lax.axis_index("subcore")
        wid = tid * NCORES + cid
        base = wid * WCHUNKS

        @pl.loop(0, RCHUNK)
        def _(r):
            for g in range(D // 16):
                rows0[pl.ds(r, 1), pl.ds(g * 16, 16)] = jnp.zeros(
                    (1, 16), jnp.float32)

        zsrc = rows0.at[pl.ds(0, RCHUNK)]
        for k in range(RITERS):
            rc = k * NSUB + tid

            @pl.when(rc < NRCHUNKS)
            def _():
                pltpu.sync_copy(zsrc, acc.at[pl.ds(rc * RCHUNK, RCHUNK)])
        plsc.subcore_barrier()

        @pl.loop(0, NSTAGES)
        def _(s):
            sbase = base + s * SCHUNKS
            pltpu.sync_copy(src_hbm.at[pl.ds(sbase, SCHUNKS)], sidx_v)
            pltpu.sync_copy(dst_hbm.at[pl.ds(sbase, SCHUNKS)], didx_v)
            pltpu.sync_copy(val_hbm.at[pl.ds(sbase, SCHUNKS)], val_v)
            pltpu.async_copy(hid_hbm.at[sidx_v.at[0]], rows0, sem0)
            pltpu.async_copy(hid_hbm.at[sidx_v.at[1]], rows1, sem1)

            @pl.loop(0, SCHUNKS // 2)
            def _(h):
                c = h * 2
                for buf, (rows_v, sem) in enumerate(((rows0, sem0),
                                                     (rows1, sem1))):
                    cc = c + buf
                    pltpu.make_async_copy(
                        hid_hbm.at[sidx_v.at[cc]], rows_v, sem).wait()
                    _scale_rows(rows_v, val_v, cc)
                    pltpu.sync_copy(rows_v, acc.at[didx_v.at[cc]], add=True)

                    @pl.when(cc + 2 < SCHUNKS)
                    def _():
                        pltpu.async_copy(
                            hid_hbm.at[sidx_v.at[cc + 2]], rows_v, sem)

        plsc.subcore_barrier()

        for k in range(RITERS):
            rc = k * NSUB + tid

            @pl.when(rc < NRCHUNKS)
            def _():
                pltpu.sync_copy(
                    acc.at[pl.ds(rc * RCHUNK, RCHUNK)],
                    part_hbm.at[cid, pl.ds(rc * RCHUNK, RCHUNK)])

    return spmm_kernel(hidden, src, dst, vals)


def _pad_chunks(a, dtype):
    a = a.astype(dtype)
    pad = jnp.zeros((PAD_EDGES - N_EDGES,), dtype)
    return jnp.concatenate([a, pad]).reshape(PCHUNKS, CHUNK)


def kernel(input, edge_index, edge_vals, W, b):
    hidden = _matmul(input, W)
    dst = _pad_chunks(edge_index[0], jnp.int32)
    src = _pad_chunks(edge_index[1], jnp.int32)
    vals = _pad_chunks(edge_vals, jnp.float32)
    partials = _spmm(hidden, src, dst, vals)
    return _combine(partials, b)
